# Initial kernel scaffold; baseline (speedup 1.0000x reference)
import jax
import jax.numpy as jnp
from jax.experimental import pallas as pl


def kernel(graph_sig, edge_index, W1, b1, W2, b2):
    raise NotImplementedError("write your pallas kernel here")


# speedup vs baseline: 17.4650x; 17.4650x over previous
import functools

import jax
import jax.numpy as jnp
from jax import lax
from jax.experimental import pallas as pl
from jax.experimental.pallas import tpu as pltpu
from jax.experimental.pallas import tpu_sc as plsc

NC, NS = 2, 16
NW = NC * NS
CHUNK = 128


def _sc_aggregate(nchunk, n, c, b_sz, acc_rows, zrows, x_flat, src3, dst3):
    rows_per_tile = acc_rows // NS
    rows_a = (n // NS) & ~7
    rows_last = n - rows_a * (NS - 1)

    mesh = plsc.VectorSubcoreMesh(core_axis_name="c", subcore_axis_name="s")

    @functools.partial(
        pl.kernel,
        out_type=jax.ShapeDtypeStruct((NC, b_sz, n, c), jnp.float32),
        mesh=mesh,
        scratch_types=[
            pltpu.VMEM((nchunk, CHUNK), jnp.int32),
            pltpu.VMEM((nchunk, CHUNK), jnp.int32),
            pltpu.VMEM((nchunk, CHUNK), jnp.int32),
            pltpu.VMEM((CHUNK, c), jnp.float32),
            pltpu.VMEM((zrows, c), jnp.float32),
            pltpu.VMEM_SHARED((acc_rows, c), jnp.float32),
            pltpu.SemaphoreType.DMA,
        ],
    )
    def agg(x_hbm, src_hbm, dst_hbm, out_hbm,
            src_v, dst_v, idxb_v, buf, zero_v, acc, gsem):
        cid = lax.axis_index("c")
        sid = lax.axis_index("s")
        wid = sid * NC + cid

        pltpu.sync_copy(src_hbm.at[wid], src_v)
        pltpu.sync_copy(dst_hbm.at[wid], dst_v)

        z16 = jnp.zeros((16,), jnp.float32)

        def zfill(r, carry):
            for i in range(c // 16):
                zero_v[r, pl.ds(i * 16, 16)] = z16
            return carry

        lax.fori_loop(0, zrows, zfill, 0)

        for b in range(b_sz):
            zbase = sid * rows_per_tile
            for t in range(rows_per_tile // zrows):
                pltpu.sync_copy(zero_v, acc.at[pl.ds(zbase + t * zrows, zrows)])
            plsc.subcore_barrier()

            bn = jnp.int32(b * n)

            def addrow(j, carry):
                for i in range(CHUNK // 16):
                    sl = pl.ds(i * 16, 16)
                    idxb_v[j, sl] = src_v[j, sl] + bn
                return carry

            lax.fori_loop(0, nchunk, addrow, 0)

            for j in range(nchunk):
                pltpu.async_copy(x_hbm.at[idxb_v.at[j]], buf, gsem).wait()
                pltpu.sync_copy(buf, acc.at[dst_v.at[j]], add=True)
            plsc.subcore_barrier()

            @pl.when(sid < NS - 1)
            def _():
                obase = sid * rows_a
                pltpu.sync_copy(acc.at[pl.ds(obase, rows_a)],
                                out_hbm.at[cid, b, pl.ds(obase, rows_a)])

            @pl.when(sid == NS - 1)
            def _():
                obase = rows_a * (NS - 1)
                pltpu.sync_copy(acc.at[pl.ds(obase, rows_last)],
                                out_hbm.at[cid, b, pl.ds(obase, rows_last)])

            plsc.subcore_barrier()

    return agg(x_flat, src3, dst3)


def _tc_affine(b_sz, n, c, blk, x, agg, w, bias):
    def body(x_ref, a_ref, w_ref, b_ref, o_ref):
        h = x_ref[0] + a_ref[0, 0] + a_ref[1, 0]
        o = lax.dot_general(h, w_ref[...], (((1,), (1,)), ((), ())),
                            preferred_element_type=jnp.float32)
        o_ref[0] = o + b_ref[...]

    return pl.pallas_call(
        body,
        grid=(b_sz, n // blk),
        in_specs=[
            pl.BlockSpec((1, blk, c), lambda i, j: (i, j, 0)),
            pl.BlockSpec((NC, 1, blk, c), lambda i, j: (0, i, j, 0)),
            pl.BlockSpec((c, c), lambda i, j: (0, 0)),
            pl.BlockSpec((1, c), lambda i, j: (0, 0)),
        ],
        out_specs=pl.BlockSpec((1, blk, c), lambda i, j: (i, j, 0)),
        out_shape=jax.ShapeDtypeStruct((b_sz, n, c), jnp.float32),
    )(x, agg, w, bias.reshape(1, c))


def kernel(graph_sig, edge_index, W1, b1, W2, b2):
    b_sz, n, c = graph_sig.shape
    e = edge_index.shape[1]

    per_tile = -(-e // (NW * CHUNK)) * CHUNK
    nchunk = per_tile // CHUNK
    ep = per_tile * NW
    acc_rows = -(-(n + 1) // (NS * 64)) * (NS * 64)
    zrows = 64

    src = edge_index[0]
    dst = edge_index[1]
    pad = ep - e
    src3 = jnp.concatenate([src, jnp.zeros((pad,), jnp.int32)]).reshape(
        NW, nchunk, CHUNK)
    dst3 = jnp.concatenate([dst, jnp.full((pad,), n, jnp.int32)]).reshape(
        NW, nchunk, CHUNK)

    x_flat = graph_sig.reshape(b_sz * n, c)
    blk = 1000 if n % 1000 == 0 else 8

    p1 = _sc_aggregate(nchunk, n, c, b_sz, acc_rows, zrows, x_flat, src3, dst3)
    h1 = _tc_affine(b_sz, n, c, blk, graph_sig, p1, W1, b1)
    p2 = _sc_aggregate(nchunk, n, c, b_sz, acc_rows, zrows,
                       h1.reshape(b_sz * n, c), src3, dst3)
    return _tc_affine(b_sz, n, c, blk, h1, p2, W2, b2)


# speedup vs baseline: 19.2877x; 1.1044x over previous
import functools

import jax
import jax.numpy as jnp
from jax import lax
from jax.experimental import pallas as pl
from jax.experimental.pallas import tpu as pltpu
from jax.experimental.pallas import tpu_sc as plsc

NC, NS = 2, 16
NW = NC * NS
CHUNK = 128


def _sc_aggregate(nchunk, n, c, b_sz, acc_rows, x_flat, src3, dst3, zrow):
    rows_per_tile = acc_rows // NS
    rows_a = (n // NS) & ~7
    rows_last = n - rows_a * (NS - 1)
    xrows_last = n - rows_per_tile * (NS - 1)

    mesh = plsc.VectorSubcoreMesh(core_axis_name="c", subcore_axis_name="s")

    @functools.partial(
        pl.kernel,
        out_type=jax.ShapeDtypeStruct((NC, b_sz, n, c), jnp.float32),
        mesh=mesh,
        scratch_types=[
            pltpu.VMEM((nchunk, CHUNK), jnp.int32),
            pltpu.VMEM((nchunk, CHUNK), jnp.int32),
            pltpu.VMEM((CHUNK, c), jnp.float32),
            pltpu.VMEM((CHUNK, c), jnp.float32),
            pltpu.VMEM_SHARED((acc_rows, c), jnp.float32),
            pltpu.SemaphoreType.DMA,
            pltpu.SemaphoreType.DMA,
        ],
    )
    def agg(x_hbm, src_hbm, dst_hbm, zrow_hbm, out_hbm,
            idx_v, dst_v, buf0, buf1, acc, gsem0, gsem1):
        cid = lax.axis_index("c")
        sid = lax.axis_index("s")
        wid = sid * NC + cid

        pltpu.sync_copy(src_hbm.at[wid], idx_v)
        pltpu.sync_copy(dst_hbm.at[wid], dst_v)

        bufs = (buf0, buf1)
        sems = (gsem0, gsem1)

        for b in range(b_sz):
            @pl.when(cid == 0)
            def _():
                @pl.when(sid < NS - 1)
                def _():
                    base = sid * rows_per_tile
                    pltpu.sync_copy(
                        x_hbm.at[pl.ds(b * n + base, rows_per_tile)],
                        acc.at[pl.ds(base, rows_per_tile)])

                @pl.when(sid == NS - 1)
                def _():
                    base = (NS - 1) * rows_per_tile
                    pltpu.sync_copy(
                        x_hbm.at[pl.ds(b * n + base, xrows_last)],
                        acc.at[pl.ds(base, xrows_last)])
                    tail = acc_rows - n
                    for t in range(tail // 64):
                        pltpu.sync_copy(
                            zrow_hbm, acc.at[pl.ds(n + t * 64, 64)])
                    rem = tail % 64
                    if rem:
                        pltpu.sync_copy(
                            zrow_hbm.at[pl.ds(0, rem)],
                            acc.at[pl.ds(n + (tail // 64) * 64, rem)])

            @pl.when(cid == 1)
            def _():
                base = sid * rows_per_tile
                for t in range(rows_per_tile // 64):
                    pltpu.sync_copy(zrow_hbm,
                                    acc.at[pl.ds(base + t * 64, 64)])

            plsc.subcore_barrier()

            xb = x_hbm.at[pl.ds(b * n, n)]
            cp = pltpu.async_copy(xb.at[idx_v.at[0]], bufs[0], sems[0])
            for j in range(nchunk):
                if j + 1 < nchunk:
                    nxt = pltpu.async_copy(xb.at[idx_v.at[j + 1]],
                                           bufs[(j + 1) % 2],
                                           sems[(j + 1) % 2])
                cp.wait()
                pltpu.sync_copy(bufs[j % 2], acc.at[dst_v.at[j]], add=True)
                if j + 1 < nchunk:
                    cp = nxt

            plsc.subcore_barrier()

            @pl.when(sid < NS - 1)
            def _():
                obase = sid * rows_a
                pltpu.sync_copy(acc.at[pl.ds(obase, rows_a)],
                                out_hbm.at[cid, b, pl.ds(obase, rows_a)])

            @pl.when(sid == NS - 1)
            def _():
                obase = rows_a * (NS - 1)
                pltpu.sync_copy(acc.at[pl.ds(obase, rows_last)],
                                out_hbm.at[cid, b, pl.ds(obase, rows_last)])

            plsc.subcore_barrier()

    return agg(x_flat, src3, dst3, zrow)


def _tc_affine(b_sz, n, c, blk, agg, w, bias):
    def body(a_ref, w_ref, b_ref, o_ref):
        h = a_ref[0, 0] + a_ref[1, 0]
        o = lax.dot_general(h, w_ref[...], (((1,), (1,)), ((), ())),
                            preferred_element_type=jnp.float32)
        o_ref[0] = o + b_ref[...]

    return pl.pallas_call(
        body,
        grid=(b_sz, n // blk),
        in_specs=[
            pl.BlockSpec((NC, 1, blk, c), lambda i, j: (0, i, j, 0)),
            pl.BlockSpec((c, c), lambda i, j: (0, 0)),
            pl.BlockSpec((1, c), lambda i, j: (0, 0)),
        ],
        out_specs=pl.BlockSpec((1, blk, c), lambda i, j: (i, j, 0)),
        out_shape=jax.ShapeDtypeStruct((b_sz, n, c), jnp.float32),
    )(agg, w, bias.reshape(1, c))


def kernel(graph_sig, edge_index, W1, b1, W2, b2):
    b_sz, n, c = graph_sig.shape
    e = edge_index.shape[1]

    per_tile = -(-e // (NW * CHUNK)) * CHUNK
    nchunk = per_tile // CHUNK
    ep = per_tile * NW
    acc_rows = -(-(n + 1) // (NS * 64)) * (NS * 64)

    src = edge_index[0]
    dst = edge_index[1]
    pad = ep - e
    srcp = jnp.concatenate([src, jnp.zeros((pad,), jnp.int32)])
    src3 = srcp.reshape(NW, nchunk, CHUNK)
    dst3 = jnp.concatenate([dst, jnp.full((pad,), n, jnp.int32)]).reshape(
        NW, nchunk, CHUNK)
    zrow = jnp.zeros((64, c), jnp.float32)

    x_flat = graph_sig.reshape(b_sz * n, c)
    blk = 1000 if n % 1000 == 0 else 8

    p1 = _sc_aggregate(nchunk, n, c, b_sz, acc_rows, x_flat, src3, dst3, zrow)
    h1 = _tc_affine(b_sz, n, c, blk, p1, W1, b1)
    p2 = _sc_aggregate(nchunk, n, c, b_sz, acc_rows,
                       h1.reshape(b_sz * n, c), src3, dst3, zrow)
    return _tc_affine(b_sz, n, c, blk, p2, W2, b2)
